# baseline (speedup 1.0000x reference)
import functools

import jax
import jax.numpy as jnp
from jax import lax
from jax.experimental import pallas as pl
from jax.experimental.pallas import tpu as pltpu
from jax.experimental.pallas import tpu_sc as plsc

N = 10000
E = 160000
G = 256
D = 300
H = 600
L = 5

WP = 384
HP = 640
CNT = 304
RP = 10240
C = 128
NSC = 2
NT = 16
NW = NSC * NT

E_CH = (E + NW * C - 1) // (NW * C)
E_PAD = E_CH * NW * C
P_CH = 8
P_PAD = P_CH * NW * C


@functools.cache
def _sc_segsum(n_out, n_cols, chunks_per_worker, nb):
  reg = (((n_out + 1) + NT - 1) // NT + 7) // 8 * 8
  rpw = n_out // NT
  K = chunks_per_worker
  assert (K - nb) % nb == 0

  def body(*refs):
    tables = refs[:n_cols]
    srcp, ldstp, zeros, out = refs[n_cols:n_cols + 4]
    srcix, ldix = refs[n_cols + 4:n_cols + 6]
    rows = refs[n_cols + 6:n_cols + 6 + nb]
    acc = refs[n_cols + 6 + nb]
    sems = refs[n_cols + 7 + nb:]
    c = lax.axis_index("c")
    s = lax.axis_index("s")
    w = s * NSC + c

    pltpu.sync_copy(srcp.at[pl.ds(w * K, K)], srcix)
    pltpu.sync_copy(ldstp.at[pl.ds(w * K, K)], ldix)

    for kp in range(n_cols):
      pltpu.sync_copy(zeros.at[pl.ds(s * reg, reg)],
                      acc.at[pl.ds(s * reg, reg)])
      plsc.subcore_barrier()

      gat = lambda j, b: pltpu.async_copy(tables[kp].at[srcix.at[j]],
                                          rows[b], sems[b])
      wat = lambda b: pltpu.make_async_copy(tables[kp].at[srcix.at[0]],
                                            rows[b], sems[b]).wait()
      for b in range(nb):
        gat(b, b)

      def outer(jo, _):
        for b in range(nb):
          j = jo * nb + b
          wat(b)
          pltpu.sync_copy(rows[b], acc.at[ldix.at[j]], add=True)
          gat(j + nb, b)
        return 0

      lax.fori_loop(0, (K - nb) // nb, outer, 0)
      for b in range(nb):
        j = K - nb + b
        wat(b)
        pltpu.sync_copy(rows[b], acc.at[ldix.at[j]], add=True)

      plsc.subcore_barrier()
      pltpu.sync_copy(acc.at[pl.ds(s * rpw, rpw)],
                      out.at[kp, c, pl.ds(s * rpw, rpw)])
      plsc.subcore_barrier()

  return pl.kernel(
      body,
      out_type=jax.ShapeDtypeStruct((n_cols, NSC, n_out, C), jnp.float32),
      mesh=plsc.VectorSubcoreMesh(core_axis_name="c", subcore_axis_name="s"),
      scratch_types=[
          pltpu.VMEM((K, C), jnp.int32),
          pltpu.VMEM((K, C), jnp.int32),
      ] + [pltpu.VMEM((C, C), jnp.float32)] * nb + [
          pltpu.VMEM_SHARED((NT * reg, C), jnp.float32),
      ] + [pltpu.SemaphoreType.DMA] * nb,
      name=f"sc_segsum_{n_out}_{n_cols}",
  )


BM = 512


def _bmix_body(an, ct, ea, ec, hist0, hist1, emb, h0, h1, h2, base1, brest):
  oh_a = (an[...] == lax.broadcasted_iota(jnp.int32, (1, 128), 1)
          ).astype(jnp.float32)
  oh_c = (ct[...] == lax.broadcasted_iota(jnp.int32, (1, 8), 1)
          ).astype(jnp.float32)
  h0v = (jnp.dot(oh_a, ea[...], preferred_element_type=jnp.float32)
         + jnp.dot(oh_c, ec[...], preferred_element_type=jnp.float32))
  h0[...] = h0v[:, 0:128]
  h1[...] = h0v[:, 128:256]
  h2[...] = h0v[:, 256:384]
  histt = hist0[...] + hist1[...]
  for i in range(L):
    bi = jnp.dot(histt, emb[i], preferred_element_type=jnp.float32)
    if i == 0:
      base1[...] = h0v + bi
    else:
      brest[i - 1, :, :] = bi


_bmix_call = pl.pallas_call(
    _bmix_body,
    grid=(RP // BM,),
    in_specs=[
        pl.BlockSpec((BM, 1), lambda i: (i, 0)),
        pl.BlockSpec((BM, 1), lambda i: (i, 0)),
        pl.BlockSpec((128, WP), lambda i: (0, 0)),
        pl.BlockSpec((8, WP), lambda i: (0, 0)),
        pl.BlockSpec((BM, 128), lambda i: (i, 0)),
        pl.BlockSpec((BM, 128), lambda i: (i, 0)),
        pl.BlockSpec((L, 128, WP), lambda i: (0, 0, 0)),
    ],
    out_specs=[pl.BlockSpec((BM, 128), lambda i: (i, 0)),
               pl.BlockSpec((BM, 128), lambda i: (i, 0)),
               pl.BlockSpec((BM, 128), lambda i: (i, 0)),
               pl.BlockSpec((BM, WP), lambda i: (i, 0)),
               pl.BlockSpec((L - 1, BM, WP), lambda i: (0, i, 0))],
    out_shape=[jax.ShapeDtypeStruct((RP, 128), jnp.float32),
               jax.ShapeDtypeStruct((RP, 128), jnp.float32),
               jax.ShapeDtypeStruct((RP, 128), jnp.float32),
               jax.ShapeDtypeStruct((RP, WP), jnp.float32),
               jax.ShapeDtypeStruct((L - 1, RP, WP), jnp.float32)],
)


def _mlp_body(p00, p01, p10, p11, p20, p21, base, w1, b1, w2, b2, g, bt,
              bnext, h0, h1, h2, basen, *, last):
  w1v = w1[...]
  z = jnp.dot(base[...], w1v, preferred_element_type=jnp.float32)
  parts = ((p00, p01), (p10, p11), (p20, p21))
  for k in range(3):
    xk = parts[k][0][...] + parts[k][1][...]
    z = z + jnp.dot(xk, w1v[128 * k:128 * (k + 1), :],
                    preferred_element_type=jnp.float32)
  z = jnp.maximum(z + b1[...], 0.0)
  z = jnp.dot(z, w2[...], preferred_element_type=jnp.float32) + b2[...]
  z = z * g[...] + bt[...]
  if not last:
    z = jnp.maximum(z, 0.0)
  h0[...] = z[:, 0:128]
  h1[...] = z[:, 128:256]
  h2[...] = z[:, 256:384]
  basen[...] = z + bnext[...]


@functools.cache
def _mlp_call(last):
  full = lambda a, b: pl.BlockSpec((a, b), lambda i: (0, 0))
  blk128 = pl.BlockSpec((BM, 128), lambda i: (i, 0))
  blkw = pl.BlockSpec((BM, WP), lambda i: (i, 0))
  return pl.pallas_call(
      functools.partial(_mlp_body, last=last),
      grid=(RP // BM,),
      in_specs=[blk128, blk128, blk128, blk128, blk128, blk128, blkw,
                full(WP, HP), full(1, HP), full(HP, WP), full(1, WP),
                full(1, WP), full(1, WP), blkw],
      out_specs=[blk128, blk128, blk128, blkw],
      out_shape=[jax.ShapeDtypeStruct((RP, 128), jnp.float32),
                 jax.ShapeDtypeStruct((RP, 128), jnp.float32),
                 jax.ShapeDtypeStruct((RP, 128), jnp.float32),
                 jax.ShapeDtypeStruct((RP, WP), jnp.float32)],
  )


def _final_body(p00, p01, p10, p11, p20, p21, wd, bd, out):
  wdv = wd[...]
  parts = ((p00, p01), (p10, p11), (p20, p21))
  p2 = p20[...] + p21[...]
  cnt = jnp.maximum(p2[:, CNT - 256:CNT - 255], 1.0)
  acc = bd[...]
  for k in range(3):
    pk = (parts[k][0][...] + parts[k][1][...]) / cnt
    acc = acc + jnp.dot(pk, wdv[128 * k:128 * (k + 1), :],
                        preferred_element_type=jnp.float32)
  out[...] = acc


_final_call = pl.pallas_call(
    _final_body,
    grid=(1,),
    in_specs=[pl.BlockSpec((G, 128), lambda i: (0, 0))] * 6
    + [pl.BlockSpec((WP, 256), lambda i: (0, 0)),
       pl.BlockSpec((1, 256), lambda i: (0, 0))],
    out_specs=pl.BlockSpec((G, 256), lambda i: (0, 0)),
    out_shape=jax.ShapeDtypeStruct((G, 256), jnp.float32),
)


def _padw(a, width=WP):
  return jnp.pad(a, ((0, 0), (0, width - a.shape[1])))


def kernel(edge_index, atomic_number, chirality_type, bond_type,
           bond_direction_type, graph_ids, params):
  f32 = jnp.float32
  i32 = jnp.int32

  dump_e = RP + jnp.arange(E_PAD - E, dtype=i32) % 128
  dump_p = G + jnp.arange(P_PAD - N, dtype=i32) % 128
  src_p = jnp.pad(edge_index[0].astype(i32),
                  (0, E_PAD - E)).reshape(E_PAD // C, C)
  dst_p = jnp.concatenate([edge_index[1].astype(i32),
                           dump_e]).reshape(E_PAD // C, C)
  combo_p = jnp.pad(
      bond_type.astype(i32) * 3 + bond_direction_type.astype(i32),
      (0, E_PAD - E)).reshape(E_PAD // C, C)
  node_p = jnp.pad(jnp.arange(N, dtype=i32),
                   (0, P_PAD - N)).reshape(P_PAD // C, C)
  gid_p = jnp.concatenate([graph_ids.astype(i32),
                           dump_p]).reshape(P_PAD // C, C)

  an = jnp.pad(atomic_number.astype(i32), (0, RP - N))[:, None]
  ct = jnp.pad(chirality_type.astype(i32), (0, RP - N))[:, None]

  co = jnp.arange(24, dtype=i32)
  onehot = jnp.concatenate(
      [(co[:, None] // 3 == jnp.arange(6)[None, :]).astype(f32),
       (co[:, None] % 3 == jnp.arange(3)[None, :]).astype(f32),
       jnp.zeros((24, 128 - 9), f32)], axis=1)
  onehot = onehot * (co[:, None] < 18).astype(f32)

  emb_atom = jnp.pad(_padw(params["emb_atom"].astype(f32)), ((0, 8), (0, 0)))
  emb_chir = jnp.pad(_padw(params["emb_chir"].astype(f32)), ((0, 5), (0, 0)))
  embcat = jnp.stack([
      jnp.concatenate([lyr["emb_bond"].astype(f32),
                       lyr["emb_bdir"].astype(f32),
                       jnp.zeros((128 - 9, D), f32)], axis=0)
      for lyr in params["layers"]])
  embcat = jnp.pad(embcat, ((0, 0), (0, 0), (0, WP - D)))

  reg_e = (((RP + 1) + NT - 1) // NT + 7) // 8 * 8
  reg_p = (((G + 1) + NT - 1) // NT + 7) // 8 * 8
  zeros_e = jnp.zeros((NT * reg_e, C), f32)
  zeros_p = jnp.zeros((NT * reg_p, C), f32)
  zeros_w = jnp.zeros((RP, WP), f32)

  seg_e3 = _sc_segsum(RP, 3, E_CH, 2)
  seg_e1 = _sc_segsum(RP, 1, E_CH, 2)
  seg_p3 = _sc_segsum(G, 3, P_CH, 2)

  hist = seg_e1(onehot, combo_p, dst_p, zeros_e)
  h0, h1, h2, base, brest = _bmix_call(an, ct, emb_atom, emb_chir,
                                       hist[0, 0], hist[0, 1], embcat)

  for i, lyr in enumerate(params["layers"]):
    last = i == L - 1
    agg = seg_e3(h0, h1, h2, src_p, dst_p, zeros_e)
    w1 = jnp.pad(_padw(lyr["W1"].astype(f32), HP), ((0, WP - D), (0, 0)))
    w2 = jnp.pad(_padw(lyr["W2"].astype(f32)), ((0, HP - H), (0, 0)))
    b1 = jnp.pad(lyr["b1"].astype(f32), (0, HP - H))[None]
    b2 = jnp.pad(lyr["b2"].astype(f32), (0, WP - D))[None]
    gm = jnp.pad(lyr["gamma"].astype(f32), (0, WP - D))[None]
    bt = jnp.pad(lyr["beta"].astype(f32), (0, WP - D))[None]
    if last:
      bt = bt.at[0, CNT].set(1.0)
      bnext = zeros_w
    else:
      bnext = brest[i]
    h0, h1, h2, base = _mlp_call(last)(
        agg[0, 0], agg[0, 1], agg[1, 0], agg[1, 1], agg[2, 0], agg[2, 1],
        base, w1, b1, w2, b2, gm, bt, bnext)

  pool = seg_p3(h0, h1, h2, node_p, gid_p, zeros_p)

  wd = jnp.pad(params["Wd"].astype(f32), ((0, WP - D), (0, 0)))
  bd = params["bd"].astype(f32)[None]
  out = _final_call(pool[0, 0], pool[0, 1], pool[1, 0], pool[1, 1],
                    pool[2, 0], pool[2, 1], wd, bd)
  return jnp.squeeze(out)


# speedup vs baseline: 1.5650x; 1.5650x over previous
import functools

import jax
import jax.numpy as jnp
from jax import lax
from jax.experimental import pallas as pl
from jax.experimental.pallas import tpu as pltpu
from jax.experimental.pallas import tpu_sc as plsc

N = 10000
E = 160000
G = 256
D = 300
H = 600
L = 5

WP = 384
HP = 640
CNT = 304
RP = 10240
C = 128
NSC = 2
NT = 16
NW = NSC * NT

E_CH = (E + NW * C - 1) // (NW * C)
E_PAD = E_CH * NW * C
P_CH = 8
P_PAD = P_CH * NW * C


@functools.cache
def _sc_segsum(n_out, n_cols, chunks_per_worker, nb):
  reg = (((n_out + 1) + NT - 1) // NT + 7) // 8 * 8
  rpw = n_out // NT
  K = chunks_per_worker
  assert (K - nb) % nb == 0

  def body(*refs):
    tables = refs[:n_cols]
    srcp, ldstp, zeros, out = refs[n_cols:n_cols + 4]
    srcix, ldix = refs[n_cols + 4:n_cols + 6]
    rows = refs[n_cols + 6:n_cols + 6 + nb]
    acc = refs[n_cols + 6 + nb]
    sems = refs[n_cols + 7 + nb:]
    c = lax.axis_index("c")
    s = lax.axis_index("s")
    w = s * NSC + c

    pltpu.sync_copy(srcp.at[pl.ds(w * K, K)], srcix)
    pltpu.sync_copy(ldstp.at[pl.ds(w * K, K)], ldix)

    for kp in range(n_cols):
      pltpu.sync_copy(zeros.at[pl.ds(s * reg, reg)],
                      acc.at[pl.ds(s * reg, reg)])
      plsc.subcore_barrier()

      gat = lambda j, b: pltpu.async_copy(tables[kp].at[srcix.at[j]],
                                          rows[b], sems[b])
      wat = lambda b: pltpu.make_async_copy(tables[kp].at[srcix.at[0]],
                                            rows[b], sems[b]).wait()
      for b in range(nb):
        gat(b, b)

      def outer(jo, _):
        for b in range(nb):
          j = jo * nb + b
          wat(b)
          pltpu.sync_copy(rows[b], acc.at[ldix.at[j]], add=True)
          gat(j + nb, b)
        return 0

      lax.fori_loop(0, (K - nb) // nb, outer, 0)
      for b in range(nb):
        j = K - nb + b
        wat(b)
        pltpu.sync_copy(rows[b], acc.at[ldix.at[j]], add=True)

      plsc.subcore_barrier()
      pltpu.sync_copy(acc.at[pl.ds(s * rpw, rpw)],
                      out.at[kp, c, pl.ds(s * rpw, rpw)])
      plsc.subcore_barrier()

  return pl.kernel(
      body,
      out_type=jax.ShapeDtypeStruct((n_cols, NSC, n_out, C), jnp.float32),
      mesh=plsc.VectorSubcoreMesh(core_axis_name="c", subcore_axis_name="s"),
      scratch_types=[
          pltpu.VMEM((K, C), jnp.int32),
          pltpu.VMEM((K, C), jnp.int32),
      ] + [pltpu.VMEM((C, C), jnp.float32)] * nb + [
          pltpu.VMEM_SHARED((NT * reg, C), jnp.float32),
      ] + [pltpu.SemaphoreType.DMA] * nb,
      name=f"sc_segsum_{n_out}_{n_cols}",
  )


BM = 512


def _bmix_body(an, ct, ea, ec, hist0, hist1, emb, h0, h1, h2, base1, brest):
  oh_a = (an[...] == lax.broadcasted_iota(jnp.int32, (1, 128), 1)
          ).astype(jnp.float32)
  oh_c = (ct[...] == lax.broadcasted_iota(jnp.int32, (1, 8), 1)
          ).astype(jnp.float32)
  h0v = (jnp.dot(oh_a, ea[...], preferred_element_type=jnp.float32)
         + jnp.dot(oh_c, ec[...], preferred_element_type=jnp.float32))
  h0[...] = h0v[:, 0:128]
  h1[...] = h0v[:, 128:256]
  h2[...] = h0v[:, 256:384]
  histt = hist0[...] + hist1[...]
  for i in range(L):
    bi = jnp.dot(histt, emb[i], preferred_element_type=jnp.float32)
    if i == 0:
      base1[...] = h0v + bi
    else:
      brest[i - 1, :, :] = bi


_bmix_call = pl.pallas_call(
    _bmix_body,
    grid=(RP // BM,),
    in_specs=[
        pl.BlockSpec((BM, 1), lambda i: (i, 0)),
        pl.BlockSpec((BM, 1), lambda i: (i, 0)),
        pl.BlockSpec((128, WP), lambda i: (0, 0)),
        pl.BlockSpec((8, WP), lambda i: (0, 0)),
        pl.BlockSpec((BM, 128), lambda i: (i, 0)),
        pl.BlockSpec((BM, 128), lambda i: (i, 0)),
        pl.BlockSpec((L, 128, WP), lambda i: (0, 0, 0)),
    ],
    out_specs=[pl.BlockSpec((BM, 128), lambda i: (i, 0)),
               pl.BlockSpec((BM, 128), lambda i: (i, 0)),
               pl.BlockSpec((BM, 128), lambda i: (i, 0)),
               pl.BlockSpec((BM, WP), lambda i: (i, 0)),
               pl.BlockSpec((L - 1, BM, WP), lambda i: (0, i, 0))],
    out_shape=[jax.ShapeDtypeStruct((RP, 128), jnp.float32),
               jax.ShapeDtypeStruct((RP, 128), jnp.float32),
               jax.ShapeDtypeStruct((RP, 128), jnp.float32),
               jax.ShapeDtypeStruct((RP, WP), jnp.float32),
               jax.ShapeDtypeStruct((L - 1, RP, WP), jnp.float32)],
)


def _mlp_body(p00, p01, p10, p11, p20, p21, base, w1, b1, w2, b2, g, bt,
              bnext, h0, h1, h2, basen, *, last):
  w1v = w1[...]
  z = jnp.dot(base[...], w1v, preferred_element_type=jnp.float32)
  parts = ((p00, p01), (p10, p11), (p20, p21))
  for k in range(3):
    xk = parts[k][0][...] + parts[k][1][...]
    z = z + jnp.dot(xk, w1v[128 * k:128 * (k + 1), :],
                    preferred_element_type=jnp.float32)
  z = jnp.maximum(z + b1[...], 0.0)
  z = jnp.dot(z, w2[...], preferred_element_type=jnp.float32) + b2[...]
  z = z * g[...] + bt[...]
  if not last:
    z = jnp.maximum(z, 0.0)
  h0[...] = z[:, 0:128]
  h1[...] = z[:, 128:256]
  h2[...] = z[:, 256:384]
  basen[...] = z + bnext[...]


@functools.cache
def _mlp_call(last):
  full = lambda a, b: pl.BlockSpec((a, b), lambda i: (0, 0))
  blk128 = pl.BlockSpec((BM, 128), lambda i: (i, 0))
  blkw = pl.BlockSpec((BM, WP), lambda i: (i, 0))
  return pl.pallas_call(
      functools.partial(_mlp_body, last=last),
      grid=(RP // BM,),
      in_specs=[blk128, blk128, blk128, blk128, blk128, blk128, blkw,
                full(WP, HP), full(1, HP), full(HP, WP), full(1, WP),
                full(1, WP), full(1, WP), blkw],
      out_specs=[blk128, blk128, blk128, blkw],
      out_shape=[jax.ShapeDtypeStruct((RP, 128), jnp.float32),
                 jax.ShapeDtypeStruct((RP, 128), jnp.float32),
                 jax.ShapeDtypeStruct((RP, 128), jnp.float32),
                 jax.ShapeDtypeStruct((RP, WP), jnp.float32)],
  )


def _pool_body(gid, h0, h1, h2, p0, p1, p2):
  i = pl.program_id(0)
  oh = (gid[...] == lax.broadcasted_iota(jnp.int32, (1, G), 1)
        ).astype(jnp.float32)
  cdims = (((0,), (0,)), ((), ()))
  for hk, pk in ((h0, p0), (h1, p1), (h2, p2)):
    part = lax.dot_general(oh, hk[...], cdims,
                           preferred_element_type=jnp.float32)
    @pl.when(i == 0)
    def _():
      pk[...] = part

    @pl.when(i > 0)
    def _():
      pk[...] = pk[...] + part


_pool_call = pl.pallas_call(
    _pool_body,
    grid=(RP // BM,),
    in_specs=[pl.BlockSpec((BM, 1), lambda i: (i, 0))]
    + [pl.BlockSpec((BM, 128), lambda i: (i, 0))] * 3,
    out_specs=[pl.BlockSpec((G, 128), lambda i: (0, 0))] * 3,
    out_shape=[jax.ShapeDtypeStruct((G, 128), jnp.float32)] * 3,
)


def _final_body(p0, p1, p2, wd, bd, out):
  wdv = wd[...]
  parts = (p0, p1, p2)
  cnt = jnp.maximum(p2[...][:, CNT - 256:CNT - 255], 1.0)
  acc = bd[...]
  for k in range(3):
    pk = parts[k][...] / cnt
    acc = acc + jnp.dot(pk, wdv[128 * k:128 * (k + 1), :],
                        preferred_element_type=jnp.float32)
  out[...] = acc


_final_call = pl.pallas_call(
    _final_body,
    grid=(1,),
    in_specs=[pl.BlockSpec((G, 128), lambda i: (0, 0))] * 3
    + [pl.BlockSpec((WP, 256), lambda i: (0, 0)),
       pl.BlockSpec((1, 256), lambda i: (0, 0))],
    out_specs=pl.BlockSpec((G, 256), lambda i: (0, 0)),
    out_shape=jax.ShapeDtypeStruct((G, 256), jnp.float32),
)


def _padw(a, width=WP):
  return jnp.pad(a, ((0, 0), (0, width - a.shape[1])))


def kernel(edge_index, atomic_number, chirality_type, bond_type,
           bond_direction_type, graph_ids, params):
  f32 = jnp.float32
  i32 = jnp.int32

  dump_e = RP + jnp.arange(E_PAD - E, dtype=i32) % 128
  src_p = jnp.pad(edge_index[0].astype(i32),
                  (0, E_PAD - E)).reshape(E_PAD // C, C)
  dst_p = jnp.concatenate([edge_index[1].astype(i32),
                           dump_e]).reshape(E_PAD // C, C)
  combo_p = jnp.pad(
      bond_type.astype(i32) * 3 + bond_direction_type.astype(i32),
      (0, E_PAD - E)).reshape(E_PAD // C, C)
  gid_rp = jnp.pad(graph_ids.astype(i32), (0, RP - N),
                   constant_values=G)[:, None]

  an = jnp.pad(atomic_number.astype(i32), (0, RP - N))[:, None]
  ct = jnp.pad(chirality_type.astype(i32), (0, RP - N))[:, None]

  co = jnp.arange(24, dtype=i32)
  onehot = jnp.concatenate(
      [(co[:, None] // 3 == jnp.arange(6)[None, :]).astype(f32),
       (co[:, None] % 3 == jnp.arange(3)[None, :]).astype(f32),
       jnp.zeros((24, 128 - 9), f32)], axis=1)
  onehot = onehot * (co[:, None] < 18).astype(f32)

  emb_atom = jnp.pad(_padw(params["emb_atom"].astype(f32)), ((0, 8), (0, 0)))
  emb_chir = jnp.pad(_padw(params["emb_chir"].astype(f32)), ((0, 5), (0, 0)))
  embcat = jnp.stack([
      jnp.concatenate([lyr["emb_bond"].astype(f32),
                       lyr["emb_bdir"].astype(f32),
                       jnp.zeros((128 - 9, D), f32)], axis=0)
      for lyr in params["layers"]])
  embcat = jnp.pad(embcat, ((0, 0), (0, 0), (0, WP - D)))

  reg_e = (((RP + 1) + NT - 1) // NT + 7) // 8 * 8
  zeros_e = jnp.zeros((NT * reg_e, C), f32)
  zeros_w = jnp.zeros((RP, WP), f32)

  seg_e3 = _sc_segsum(RP, 3, E_CH, 2)
  seg_e1 = _sc_segsum(RP, 1, E_CH, 2)

  hist = seg_e1(onehot, combo_p, dst_p, zeros_e)
  h0, h1, h2, base, brest = _bmix_call(an, ct, emb_atom, emb_chir,
                                       hist[0, 0], hist[0, 1], embcat)

  for i, lyr in enumerate(params["layers"]):
    last = i == L - 1
    agg = seg_e3(h0, h1, h2, src_p, dst_p, zeros_e)
    w1 = jnp.pad(_padw(lyr["W1"].astype(f32), HP), ((0, WP - D), (0, 0)))
    w2 = jnp.pad(_padw(lyr["W2"].astype(f32)), ((0, HP - H), (0, 0)))
    b1 = jnp.pad(lyr["b1"].astype(f32), (0, HP - H))[None]
    b2 = jnp.pad(lyr["b2"].astype(f32), (0, WP - D))[None]
    gm = jnp.pad(lyr["gamma"].astype(f32), (0, WP - D))[None]
    bt = jnp.pad(lyr["beta"].astype(f32), (0, WP - D))[None]
    if last:
      bt = bt.at[0, CNT].set(1.0)
      bnext = zeros_w
    else:
      bnext = brest[i]
    h0, h1, h2, base = _mlp_call(last)(
        agg[0, 0], agg[0, 1], agg[1, 0], agg[1, 1], agg[2, 0], agg[2, 1],
        base, w1, b1, w2, b2, gm, bt, bnext)

  p0, p1, p2 = _pool_call(gid_rp, h0, h1, h2)

  wd = jnp.pad(params["Wd"].astype(f32), ((0, WP - D), (0, 0)))
  bd = params["bd"].astype(f32)[None]
  out = _final_call(p0, p1, p2, wd, bd)
  return jnp.squeeze(out)


# speedup vs baseline: 1.5655x; 1.0003x over previous
import functools

import jax
import jax.numpy as jnp
from jax import lax
from jax.experimental import pallas as pl
from jax.experimental.pallas import tpu as pltpu
from jax.experimental.pallas import tpu_sc as plsc

N = 10000
E = 160000
G = 256
D = 300
H = 600
L = 5

WP = 384
HP = 640
CNT = 304
RP = 10240
C = 128
NSC = 2
NT = 16
NW = NSC * NT

E_CH = (E + NW * C - 1) // (NW * C)
E_PAD = E_CH * NW * C


@functools.cache
def _sc_segsum(n_out, n_cols, chunks_per_worker, nb):
  reg = (((n_out + 1) + NT - 1) // NT + 7) // 8 * 8
  rpw = n_out // NT
  K = chunks_per_worker
  assert (K - nb) % nb == 0

  def body(*refs):
    tables = refs[:n_cols]
    srcp, ldstp, zeros, out = refs[n_cols:n_cols + 4]
    srcix, ldix = refs[n_cols + 4:n_cols + 6]
    rows = refs[n_cols + 6:n_cols + 6 + nb]
    acc = refs[n_cols + 6 + nb]
    sems = refs[n_cols + 7 + nb:]
    c = lax.axis_index("c")
    s = lax.axis_index("s")
    w = s * NSC + c

    pltpu.sync_copy(srcp.at[pl.ds(w * K, K)], srcix)
    pltpu.sync_copy(ldstp.at[pl.ds(w * K, K)], ldix)

    for kp in range(n_cols):
      pltpu.sync_copy(zeros.at[pl.ds(s * reg, reg)],
                      acc.at[pl.ds(s * reg, reg)])
      plsc.subcore_barrier()

      gat = lambda j, b: pltpu.async_copy(tables[kp].at[srcix.at[j]],
                                          rows[b], sems[b])
      wat = lambda b: pltpu.make_async_copy(tables[kp].at[srcix.at[0]],
                                            rows[b], sems[b]).wait()
      for b in range(nb):
        gat(b, b)

      def outer(jo, _):
        for b in range(nb):
          j = jo * nb + b
          wat(b)
          pltpu.sync_copy(rows[b], acc.at[ldix.at[j]], add=True)
          gat(j + nb, b)
        return 0

      lax.fori_loop(0, (K - nb) // nb, outer, 0)
      for b in range(nb):
        j = K - nb + b
        wat(b)
        pltpu.sync_copy(rows[b], acc.at[ldix.at[j]], add=True)

      plsc.subcore_barrier()
      pltpu.sync_copy(acc.at[pl.ds(s * rpw, rpw)],
                      out.at[kp, c, pl.ds(s * rpw, rpw)])
      plsc.subcore_barrier()

  return pl.kernel(
      body,
      out_type=jax.ShapeDtypeStruct((n_cols, NSC, n_out, C), jnp.float32),
      mesh=plsc.VectorSubcoreMesh(core_axis_name="c", subcore_axis_name="s"),
      scratch_types=[
          pltpu.VMEM((K, C), jnp.int32),
          pltpu.VMEM((K, C), jnp.int32),
      ] + [pltpu.VMEM((C, C), jnp.float32)] * nb + [
          pltpu.VMEM_SHARED((NT * reg, C), jnp.float32),
      ] + [pltpu.SemaphoreType.DMA] * nb,
      name=f"sc_segsum_{n_out}_{n_cols}",
  )


BM = 512


def _bmix_body(an, ct, ea, ec, hist0, hist1, emb, h0, h1, h2, base1, brest):
  oh_a = (an[...] == lax.broadcasted_iota(jnp.int32, (1, 128), 1)
          ).astype(jnp.float32)
  oh_c = (ct[...] == lax.broadcasted_iota(jnp.int32, (1, 8), 1)
          ).astype(jnp.float32)
  h0v = (jnp.dot(oh_a, ea[...], preferred_element_type=jnp.float32)
         + jnp.dot(oh_c, ec[...], preferred_element_type=jnp.float32))
  h0[...] = h0v[:, 0:128]
  h1[...] = h0v[:, 128:256]
  h2[...] = h0v[:, 256:384]
  histt = hist0[...] + hist1[...]
  for i in range(L):
    bi = jnp.dot(histt, emb[i], preferred_element_type=jnp.float32)
    if i == 0:
      base1[...] = h0v + bi
    else:
      brest[i - 1, :, :] = bi


_bmix_call = pl.pallas_call(
    _bmix_body,
    grid=(RP // BM,),
    in_specs=[
        pl.BlockSpec((BM, 1), lambda i: (i, 0)),
        pl.BlockSpec((BM, 1), lambda i: (i, 0)),
        pl.BlockSpec((128, WP), lambda i: (0, 0)),
        pl.BlockSpec((8, WP), lambda i: (0, 0)),
        pl.BlockSpec((BM, 128), lambda i: (i, 0)),
        pl.BlockSpec((BM, 128), lambda i: (i, 0)),
        pl.BlockSpec((L, 128, WP), lambda i: (0, 0, 0)),
    ],
    out_specs=[pl.BlockSpec((BM, 128), lambda i: (i, 0)),
               pl.BlockSpec((BM, 128), lambda i: (i, 0)),
               pl.BlockSpec((BM, 128), lambda i: (i, 0)),
               pl.BlockSpec((BM, WP), lambda i: (i, 0)),
               pl.BlockSpec((L - 1, BM, WP), lambda i: (0, i, 0))],
    out_shape=[jax.ShapeDtypeStruct((RP, 128), jnp.float32),
               jax.ShapeDtypeStruct((RP, 128), jnp.float32),
               jax.ShapeDtypeStruct((RP, 128), jnp.float32),
               jax.ShapeDtypeStruct((RP, WP), jnp.float32),
               jax.ShapeDtypeStruct((L - 1, RP, WP), jnp.float32)],
)


def _mlp_body(p00, p01, p10, p11, p20, p21, base, w1, b1, w2, b2, g, bt,
              bnext, h0, h1, h2, basen, *, last):
  w1v = w1[...]
  z = jnp.dot(base[...], w1v, preferred_element_type=jnp.float32)
  parts = ((p00, p01), (p10, p11), (p20, p21))
  for k in range(3):
    xk = parts[k][0][...] + parts[k][1][...]
    z = z + jnp.dot(xk, w1v[128 * k:128 * (k + 1), :],
                    preferred_element_type=jnp.float32)
  z = jnp.maximum(z + b1[...], 0.0)
  z = jnp.dot(z, w2[...], preferred_element_type=jnp.float32) + b2[...]
  z = z * g[...] + bt[...]
  if not last:
    z = jnp.maximum(z, 0.0)
  h0[...] = z[:, 0:128]
  h1[...] = z[:, 128:256]
  h2[...] = z[:, 256:384]
  basen[...] = z + bnext[...]


@functools.cache
def _mlp_call(last):
  full = lambda a, b: pl.BlockSpec((a, b), lambda i: (0, 0))
  blk128 = pl.BlockSpec((BM, 128), lambda i: (i, 0))
  blkw = pl.BlockSpec((BM, WP), lambda i: (i, 0))
  return pl.pallas_call(
      functools.partial(_mlp_body, last=last),
      grid=(RP // BM,),
      in_specs=[blk128, blk128, blk128, blk128, blk128, blk128, blkw,
                full(WP, HP), full(1, HP), full(HP, WP), full(1, WP),
                full(1, WP), full(1, WP), blkw],
      out_specs=[blk128, blk128, blk128, blkw],
      out_shape=[jax.ShapeDtypeStruct((RP, 128), jnp.float32),
                 jax.ShapeDtypeStruct((RP, 128), jnp.float32),
                 jax.ShapeDtypeStruct((RP, 128), jnp.float32),
                 jax.ShapeDtypeStruct((RP, WP), jnp.float32)],
  )


def _pool_body(gid, h0, h1, h2, p0, p1, p2):
  i = pl.program_id(0)
  oh = (gid[...] == lax.broadcasted_iota(jnp.int32, (1, G), 1)
        ).astype(jnp.float32)
  cdims = (((0,), (0,)), ((), ()))
  for hk, pk in ((h0, p0), (h1, p1), (h2, p2)):
    part = lax.dot_general(oh, hk[...], cdims,
                           preferred_element_type=jnp.float32)
    @pl.when(i == 0)
    def _():
      pk[...] = part

    @pl.when(i > 0)
    def _():
      pk[...] = pk[...] + part


_pool_call = pl.pallas_call(
    _pool_body,
    grid=(RP // BM,),
    in_specs=[pl.BlockSpec((BM, 1), lambda i: (i, 0))]
    + [pl.BlockSpec((BM, 128), lambda i: (i, 0))] * 3,
    out_specs=[pl.BlockSpec((G, 128), lambda i: (0, 0))] * 3,
    out_shape=[jax.ShapeDtypeStruct((G, 128), jnp.float32)] * 3,
)


def _final_body(p0, p1, p2, wd, bd, out):
  wdv = wd[...]
  parts = (p0, p1, p2)
  cnt = jnp.maximum(p2[...][:, CNT - 256:CNT - 255], 1.0)
  acc = bd[...]
  for k in range(3):
    pk = parts[k][...] / cnt
    acc = acc + jnp.dot(pk, wdv[128 * k:128 * (k + 1), :],
                        preferred_element_type=jnp.float32)
  out[...] = acc


_final_call = pl.pallas_call(
    _final_body,
    grid=(1,),
    in_specs=[pl.BlockSpec((G, 128), lambda i: (0, 0))] * 3
    + [pl.BlockSpec((WP, 256), lambda i: (0, 0)),
       pl.BlockSpec((1, 256), lambda i: (0, 0))],
    out_specs=pl.BlockSpec((G, 256), lambda i: (0, 0)),
    out_shape=jax.ShapeDtypeStruct((G, 256), jnp.float32),
)


def _padw(a, width=WP):
  return jnp.pad(a, ((0, 0), (0, width - a.shape[1])))


def kernel(edge_index, atomic_number, chirality_type, bond_type,
           bond_direction_type, graph_ids, params):
  f32 = jnp.float32
  i32 = jnp.int32

  dump_e = RP + jnp.arange(E_PAD - E, dtype=i32) % 128
  src_p = jnp.pad(edge_index[0].astype(i32),
                  (0, E_PAD - E)).reshape(E_PAD // C, C)
  dst_p = jnp.concatenate([edge_index[1].astype(i32),
                           dump_e]).reshape(E_PAD // C, C)
  combo_p = jnp.pad(
      bond_type.astype(i32) * 3 + bond_direction_type.astype(i32),
      (0, E_PAD - E)).reshape(E_PAD // C, C)
  gid_rp = jnp.pad(graph_ids.astype(i32), (0, RP - N),
                   constant_values=G)[:, None]

  an = jnp.pad(atomic_number.astype(i32), (0, RP - N))[:, None]
  ct = jnp.pad(chirality_type.astype(i32), (0, RP - N))[:, None]

  co = jnp.arange(24, dtype=i32)
  onehot = jnp.concatenate(
      [(co[:, None] // 3 == jnp.arange(6)[None, :]).astype(f32),
       (co[:, None] % 3 == jnp.arange(3)[None, :]).astype(f32),
       jnp.zeros((24, 128 - 9), f32)], axis=1)
  onehot = onehot * (co[:, None] < 18).astype(f32)

  emb_atom = jnp.pad(_padw(params["emb_atom"].astype(f32)), ((0, 8), (0, 0)))
  emb_chir = jnp.pad(_padw(params["emb_chir"].astype(f32)), ((0, 5), (0, 0)))
  embcat = jnp.stack([
      jnp.concatenate([lyr["emb_bond"].astype(f32),
                       lyr["emb_bdir"].astype(f32),
                       jnp.zeros((128 - 9, D), f32)], axis=0)
      for lyr in params["layers"]])
  embcat = jnp.pad(embcat, ((0, 0), (0, 0), (0, WP - D)))

  reg_e = (((RP + 1) + NT - 1) // NT + 7) // 8 * 8
  zeros_e = jnp.zeros((NT * reg_e, C), f32)
  zeros_w = jnp.zeros((RP, WP), f32)

  seg_e3 = _sc_segsum(RP, 3, E_CH, 2)
  seg_e1 = _sc_segsum(RP, 1, E_CH, 2)

  hist = seg_e1(onehot, combo_p, dst_p, zeros_e)
  h0, h1, h2, base, brest = _bmix_call(an, ct, emb_atom, emb_chir,
                                       hist[0, 0], hist[0, 1], embcat)

  for i, lyr in enumerate(params["layers"]):
    last = i == L - 1
    agg = seg_e3(h0, h1, h2, src_p, dst_p, zeros_e)
    w1 = jnp.pad(_padw(lyr["W1"].astype(f32), HP), ((0, WP - D), (0, 0)))
    w2 = jnp.pad(_padw(lyr["W2"].astype(f32)), ((0, HP - H), (0, 0)))
    b1 = jnp.pad(lyr["b1"].astype(f32), (0, HP - H))[None]
    b2 = jnp.pad(lyr["b2"].astype(f32), (0, WP - D))[None]
    gm = jnp.pad(lyr["gamma"].astype(f32), (0, WP - D))[None]
    bt = jnp.pad(lyr["beta"].astype(f32), (0, WP - D))[None]
    if last:
      bt = bt.at[0, CNT].set(1.0)
      bnext = zeros_w
    else:
      bnext = brest[i]
    h0, h1, h2, base = _mlp_call(last)(
        agg[0, 0], agg[0, 1], agg[1, 0], agg[1, 1], agg[2, 0], agg[2, 1],
        base, w1, b1, w2, b2, gm, bt, bnext)

  p0, p1, p2 = _pool_call(gid_rp, h0, h1, h2)

  wd = jnp.pad(params["Wd"].astype(f32), ((0, WP - D), (0, 0)))
  bd = params["bd"].astype(f32)[None]
  out = _final_call(p0, p1, p2, wd, bd)
  return jnp.squeeze(out)
